# baseline (speedup 1.0000x reference)
import functools

import jax
import jax.numpy as jnp
from jax import lax
from jax.experimental import pallas as pl
from jax.experimental.pallas import tpu as pltpu
from jax.experimental.pallas import tpu_sc as plsc

_P = 16
_NH = 24
_N = _NH * _NH
_C = 3


def _mask_body(x_ref, m_ref):
    t = x_ref[0].reshape(_C, _NH, _P, _NH, _P)
    mx = jnp.max(jnp.max(jnp.max(t, axis=4), axis=2), axis=0)
    m_ref[0] = (mx > 0.0).astype(jnp.int32).reshape(1, _N)


def _sc_body(masks_hbm, xr_hbm, out_hbm, mv, fsrc, rbase, colb, idxb,
             inbuf, outbuf, sem):
    s = lax.axis_index("s")
    h = lax.axis_index("c")
    b = s
    iota = lax.broadcasted_iota(jnp.int32, (16,), 0)

    pltpu.sync_copy(masks_hbm.at[b], mv)

    def count_step(k, acc):
        return acc + jnp.sum(mv[pl.ds(16 * k, 16)].astype(jnp.float32))
    c_totf = lax.fori_loop(0, _N // 16, count_step, jnp.float32(0))
    c_tot = c_totf.astype(jnp.int32)

    def perm_step(k, kept):
        m16f = mv[pl.ds(16 * k, 16)].astype(jnp.float32)
        cs = plsc.cumsum(m16f)
        psum_ex = kept + cs - m16f
        n = 16 * k + iota
        nf = n.astype(jnp.float32)
        destf = jnp.where(m16f > 0, psum_ex, c_totf + nf - psum_ex)
        plsc.store_scatter(fsrc, [destf.astype(jnp.int32)], n)
        return kept + jnp.sum(m16f)
    lax.fori_loop(0, _N // 16, perm_step, jnp.float32(0))

    def base_step(k, _):
        f16 = fsrc[pl.ds(16 * k, 16)]
        rbase[pl.ds(16 * k, 16)] = f16 // _NH
        colb[pl.ds(16 * k, 16)] = _P * (f16 % _NH)
        return 0
    lax.fori_loop(0, _N // 16, base_step, 0)

    def block_step(g, _):
        i = 8 * h + g
        for t in range(4):
            l = 16 * t + iota
            idxb[pl.ds(16 * t, 16)] = (_C * b + l // _NH) * 384 + _P * (l % _NH) + i
        l = 56 + iota
        idxb[pl.ds(56, 16)] = (_C * b + l // _NH) * 384 + _P * (l % _NH) + i
        pltpu.async_copy(xr_hbm.at[idxb], inbuf, sem).wait()

        def row_step(rr, _r):
            cpl = rr // _P
            j = rr % _P
            for g2 in range(_N // 16):
                ri = rbase[pl.ds(16 * g2, 16)] + _NH * cpl
                ci = colb[pl.ds(16 * g2, 16)] + j
                v = plsc.load_gather(inbuf, [ri, ci])
                keep = (16 * g2 + iota) < c_tot
                outbuf[rr, pl.ds(16 * g2, 16)] = jnp.where(keep, v, 0.0)
            return 0
        lax.fori_loop(0, _C * _P, row_step, 0)

        pltpu.sync_copy(
            outbuf, out_hbm.at[pl.ds((_P * b + i) * _C * _P, _C * _P)])
        return 0
    lax.fori_loop(0, _P // 2, block_step, 0)


def _sc_compact(masks, xr):
    mesh = plsc.VectorSubcoreMesh(
        core_axis_name="c", subcore_axis_name="s", num_cores=2,
        num_subcores=16)
    B = masks.shape[0]
    run = functools.partial(
        pl.kernel,
        out_type=jax.ShapeDtypeStruct((B * _P * _C * _P, _N), jnp.float32),
        mesh=mesh,
        scratch_types=[
            pltpu.VMEM((_N,), jnp.int32),
            pltpu.VMEM((_N,), jnp.int32),
            pltpu.VMEM((_N,), jnp.int32),
            pltpu.VMEM((_N,), jnp.int32),
            pltpu.VMEM((72,), jnp.int32),
            pltpu.VMEM((72, 384), jnp.float32),
            pltpu.VMEM((_C * _P, _N), jnp.float32),
            pltpu.SemaphoreType.DMA,
        ],
        compiler_params=pltpu.CompilerParams(needs_layout_passes=False),
    )(_sc_body)
    return run(masks, xr)


def kernel(images):
    B, H, W, C = images.shape
    x4 = images.transpose(0, 3, 1, 2)
    masks3 = pl.pallas_call(
        _mask_body,
        grid=(B,),
        in_specs=[pl.BlockSpec((1, C, H, W), lambda i: (i, 0, 0, 0))],
        out_specs=pl.BlockSpec((1, 1, _N), lambda i: (i, 0, 0)),
        out_shape=jax.ShapeDtypeStruct((B, 1, _N), jnp.int32),
    )(x4)
    masks = masks3.reshape(B, _N)
    xr = x4.reshape(B * C * H, W)
    out_t = _sc_compact(masks, xr)
    out5t = out_t.reshape(B, _P, C, _P, _N)
    return out5t.transpose(0, 4, 1, 3, 2)


# speedup vs baseline: 1.2531x; 1.2531x over previous
import functools

import jax
import jax.numpy as jnp
from jax import lax
from jax.experimental import pallas as pl
from jax.experimental.pallas import tpu as pltpu
from jax.experimental.pallas import tpu_sc as plsc

_P = 16
_NH = 24
_N = _NH * _NH
_C = 3


def _mask_body(x_ref, m_ref):
    t = x_ref[0].reshape(_C, _NH, _P, _NH, _P)
    mx = jnp.max(jnp.max(jnp.max(t, axis=4), axis=2), axis=0)
    m_ref[0] = (mx > 0.0).astype(jnp.int32).reshape(1, _N)


_PITCH = _N + 1


def _sc_body(masks_hbm, xr_hbm, out_hbm, mv, destd, idxb, inbuf, outbuf,
             outc, sem, wsem):
    s = lax.axis_index("s")
    h = lax.axis_index("c")
    b = s
    iota = lax.broadcasted_iota(jnp.int32, (16,), 0)

    pltpu.sync_copy(masks_hbm.at[b], mv)

    def count_step(k, acc):
        return acc + jnp.sum(mv[pl.ds(16 * k, 16)].astype(jnp.float32))
    c_totf = lax.fori_loop(0, _N // 16, count_step, jnp.float32(0))
    c_tot = c_totf.astype(jnp.int32)

    def perm_step(k, kept):
        m16f = mv[pl.ds(16 * k, 16)].astype(jnp.float32)
        cs = plsc.cumsum(m16f)
        psum_ex = kept + cs - m16f
        n = 16 * k + iota
        nf = n.astype(jnp.float32)
        destf = jnp.where(m16f > 0, psum_ex, c_totf + nf - psum_ex)
        d = destf.astype(jnp.int32)
        sv = jnp.where(m16f > 0, d, -d - 1)
        pos = 32 * (n // _NH) + n % _NH
        plsc.store_scatter(destd, [pos], sv)
        return kept + jnp.sum(m16f)
    lax.fori_loop(0, _N // 16, perm_step, jnp.float32(0))

    def block_step(g, _):
        i = 8 * h + g
        for t in range(4):
            l = 16 * t + iota
            idxb[pl.ds(16 * t, 16)] = (_C * b + l // _NH) * 384 + _P * (l % _NH) + i
        l = 56 + iota
        idxb[pl.ds(56, 16)] = (_C * b + l // _NH) * 384 + _P * (l % _NH) + i
        pltpu.async_copy(xr_hbm.at[idxb], inbuf, sem).wait()

        def row_step(rr, _r):
            cpl = rr // _NH
            r = rr % _NH
            dv1 = destd[pl.ds(32 * r, 16)]
            dv2 = destd[pl.ds(32 * r + 16, 16)]
            rowidx = cpl * _P + iota
            for cc in range(_NH):
                d = dv1[cc] if cc < 16 else dv2[cc - 16]

                @pl.when(d >= 0)
                def _put():
                    v = inbuf[rr, pl.ds(_P * cc, 16)]
                    plsc.store_scatter(outbuf, [rowidx, iota * 0 + d], v)
            return 0
        lax.fori_loop(0, _C * _NH, row_step, 0)

        def crow(rr, _z):
            for gz in range(_N // 16):
                v = outbuf[rr, pl.ds(16 * gz, 16)]
                keep = (16 * gz + iota) < c_tot
                outc[rr, pl.ds(16 * gz, 16)] = jnp.where(keep, v, 0.0)
            return 0
        lax.fori_loop(0, _C * _P, crow, 0)

        base = (_P * b + i) * _C * _P
        pltpu.async_copy(outc, out_hbm.at[pl.ds(base, _C * _P)], wsem).wait()
        return 0
    lax.fori_loop(0, _P // 2, block_step, 0)


def _sc_compact(masks, xr):
    mesh = plsc.VectorSubcoreMesh(
        core_axis_name="c", subcore_axis_name="s", num_cores=2,
        num_subcores=16)
    B = masks.shape[0]
    run = functools.partial(
        pl.kernel,
        out_type=jax.ShapeDtypeStruct((B * _P * _C * _P, _N), jnp.float32),
        mesh=mesh,
        scratch_types=[
            pltpu.VMEM((_N,), jnp.int32),
            pltpu.VMEM((2 * 384,), jnp.int32),
            pltpu.VMEM((72,), jnp.int32),
            pltpu.VMEM((72, 384), jnp.float32),
            pltpu.VMEM((_C * _P, _PITCH), jnp.float32),
            pltpu.VMEM((_C * _P, _N), jnp.float32),
            pltpu.SemaphoreType.DMA,
            pltpu.SemaphoreType.DMA,
        ],
        compiler_params=pltpu.CompilerParams(needs_layout_passes=False),
    )(_sc_body)
    return run(masks, xr)


def kernel(images):
    B, H, W, C = images.shape
    x4 = images.transpose(0, 3, 1, 2)
    masks3 = pl.pallas_call(
        _mask_body,
        grid=(B,),
        in_specs=[pl.BlockSpec((1, C, H, W), lambda i: (i, 0, 0, 0))],
        out_specs=pl.BlockSpec((1, 1, _N), lambda i: (i, 0, 0)),
        out_shape=jax.ShapeDtypeStruct((B, 1, _N), jnp.int32),
    )(x4)
    masks = masks3.reshape(B, _N)
    xr = x4.reshape(B * C * H, W)
    out_t = _sc_compact(masks, xr)
    out5t = out_t.reshape(B, _P, C, _P, _N)
    return out5t.transpose(0, 4, 1, 3, 2)


# speedup vs baseline: 2.0901x; 1.6679x over previous
import jax
import jax.numpy as jnp
from jax import lax
from jax.experimental import pallas as pl


def _body(x_ref, o_ref):
    n_h, n_w, p, pc = 24, 24, 16, 48
    N, D = n_h * n_w, p * pc
    xb = x_ref[0]
    patches = xb.reshape(n_h, p, n_w, pc).transpose(0, 2, 1, 3).reshape(N, D)
    mx = jnp.max(patches, axis=1, keepdims=True)
    mask = mx > 0.0
    maskf = mask.astype(jnp.float32)
    count = jnp.sum(maskf)

    def fast(_):
        return patches

    def slow(_):
        row = lax.broadcasted_iota(jnp.int32, (N, N), 0)
        col = lax.broadcasted_iota(jnp.int32, (N, N), 1).astype(jnp.float32)
        tri = (col < row.astype(jnp.float32)).astype(jnp.float32)
        psum_ex = jnp.dot(tri, maskf, preferred_element_type=jnp.float32)
        n_idx = lax.broadcasted_iota(jnp.int32, (N, 1), 0).astype(jnp.float32)
        dest = jnp.where(mask, psum_ex, count + (n_idx - psum_ex))
        onehot_t = (dest == col).astype(jnp.float32)
        masked = patches * maskf
        return lax.dot_general(
            onehot_t, masked, (((0,), (0,)), ((), ())),
            preferred_element_type=jnp.float32)

    o_ref[0] = lax.cond(count == float(N), fast, slow, None)


def kernel(images):
    B, H, W, C = images.shape
    p = 16
    n_h, n_w = H // p, W // p
    N, D = n_h * n_w, p * p * C
    x = images.reshape(B, H, W * C)
    out = pl.pallas_call(
        _body,
        grid=(B,),
        in_specs=[pl.BlockSpec((1, H, W * C), lambda b: (b, 0, 0))],
        out_specs=pl.BlockSpec((1, N, D), lambda b: (b, 0, 0)),
        out_shape=jax.ShapeDtypeStruct((B, N, D), jnp.float32),
    )(x)
    return out.reshape(B, N, p, p, C)


# speedup vs baseline: 2.2197x; 1.0620x over previous
import jax
import jax.numpy as jnp
from jax import lax
from jax.experimental import pallas as pl

_P = 16
_NH = 24
_N = _NH * _NH
_C = 3
_R = _P * _P * _C


def _body(x_ref, o_ref):
    t = x_ref[0].reshape(_C, _NH, _P, _NH, _P)
    o = t.transpose(2, 0, 4, 1, 3).reshape(_R, _N)
    mx = jnp.max(o, axis=0, keepdims=True)
    mask = mx > 0.0
    maskf = mask.astype(jnp.float32)
    count = jnp.sum(maskf)

    def fast(_):
        return o

    def slow(_):
        rowi = lax.broadcasted_iota(jnp.int32, (_N, _N), 0).astype(jnp.float32)
        coli = lax.broadcasted_iota(jnp.int32, (_N, _N), 1).astype(jnp.float32)
        tri_u = (rowi < coli).astype(jnp.float32)
        psum_ex = jnp.dot(maskf, tri_u, preferred_element_type=jnp.float32)
        n_idx = lax.broadcasted_iota(jnp.int32, (1, _N), 1).astype(jnp.float32)
        dest = jnp.where(mask, psum_ex, count + (n_idx - psum_ex))
        onehot = (dest == lax.broadcasted_iota(
            jnp.int32, (_N, _N), 0).astype(jnp.float32)).astype(jnp.float32)
        masked = o * maskf
        return lax.dot_general(
            masked, onehot, (((1,), (1,)), ((), ())),
            preferred_element_type=jnp.float32)

    o_ref[0] = lax.cond(count == float(_N), fast, slow, None)


def kernel(images):
    B, H, W, C = images.shape
    x4 = images.transpose(0, 3, 1, 2)
    out_t = pl.pallas_call(
        _body,
        grid=(B,),
        in_specs=[pl.BlockSpec((1, C, H, W), lambda b: (b, 0, 0, 0))],
        out_specs=pl.BlockSpec((1, _R, _N), lambda b: (b, 0, 0)),
        out_shape=jax.ShapeDtypeStruct((B, _R, _N), jnp.float32),
    )(x4)
    out5t = out_t.reshape(B, _P, C, _P, _N)
    return out5t.transpose(0, 4, 1, 3, 2)
